# baseline (speedup 1.0000x reference)
import jax
import jax.numpy as jnp
from jax.experimental import pallas as pl
from jax.experimental.pallas import tpu as pltpu

_N = 10000
_D = 128
_BM = 400
_NB = _N // _BM
_DEPTH = 3


def _proj_body(feat_ref, w1_ref, g_ref):
    f = feat_ref[...].astype(jnp.bfloat16)
    w = w1_ref[...].astype(jnp.bfloat16)
    g = jax.lax.dot_general(
        f, w, (((1,), (0,)), ((), ())), preferred_element_type=jnp.float32
    )
    g_ref[...] = g.astype(jnp.bfloat16)


def _dot(a, b):
    return jax.lax.dot_general(
        a, b, (((1,), (0,)), ((), ())), preferred_element_type=jnp.float32
    )


def _fused_body(
    adj_hbm, g_ref, b1_ref, w2_ref, out_ref, ab0, ab1, ab2, h2_ref, sems
):
    total = 2 * _NB
    abufs = (ab0, ab1, ab2)

    def _copy(t, s):
        blk = jax.lax.rem(t, _NB)
        return pltpu.make_async_copy(
            adj_hbm.at[pl.ds(blk * _BM, _BM), :],
            abufs[s],
            sems.at[s],
        )

    for t in range(_DEPTH - 1):
        _copy(t, t % _DEPTH).start()

    def _step(t, carry):
        slot = jax.lax.rem(t, _DEPTH)
        blk = jax.lax.rem(t, _NB)

        for s in range(_DEPTH):

            @pl.when(slot == s)
            def _slot_work(s=s):
                _copy(t, s).wait()

                @pl.when(t + _DEPTH - 1 < total)
                def _prefetch():
                    _copy(t + _DEPTH - 1, (s + _DEPTH - 1) % _DEPTH).start()

                a_ref = abufs[s]

                @pl.when(t < _NB)
                def _phase0():
                    acc = _dot(a_ref[...], g_ref[...])
                    h = jnp.tanh(acc + b1_ref[...])
                    h2 = _dot(
                        h.astype(jnp.bfloat16),
                        w2_ref[...].astype(jnp.bfloat16),
                    )
                    h2_ref[pl.ds(blk * _BM, _BM), :] = h2.astype(jnp.bfloat16)

                @pl.when(t >= _NB)
                def _phase1():
                    out_ref[pl.ds(blk * _BM, _BM), :] = _dot(
                        a_ref[...], h2_ref[...]
                    )

        return carry

    jax.lax.fori_loop(0, total, _step, 0)


@jax.jit
def _run(adj, feat, W1, b1, W2):
    n, d = _N, _D

    g = pl.pallas_call(
        _proj_body,
        out_shape=jax.ShapeDtypeStruct((n, d), jnp.bfloat16),
    )(feat, W1)

    b1_2d = b1.reshape(1, d)

    out = pl.pallas_call(
        _fused_body,
        in_specs=[
            pl.BlockSpec(memory_space=pl.ANY),
            pl.BlockSpec(memory_space=pltpu.VMEM),
            pl.BlockSpec(memory_space=pltpu.VMEM),
            pl.BlockSpec(memory_space=pltpu.VMEM),
        ],
        out_specs=pl.BlockSpec(memory_space=pltpu.VMEM),
        out_shape=jax.ShapeDtypeStruct((n, d), jnp.float32),
        scratch_shapes=[
            pltpu.VMEM((_BM, n), jnp.float32),
            pltpu.VMEM((_BM, n), jnp.float32),
            pltpu.VMEM((_BM, n), jnp.float32),
            pltpu.VMEM((n, d), jnp.bfloat16),
            pltpu.SemaphoreType.DMA((_DEPTH,)),
        ],
        compiler_params=pltpu.CompilerParams(
            vmem_limit_bytes=63 * 1024 * 1024,
        ),
    )(adj, g, b1_2d, W2)
    return out


def kernel(adj, feat, W1, b1, W2):
    return _run(adj, feat, W1, b1, W2)


# speedup vs baseline: 1.0336x; 1.0336x over previous
import functools

import jax
import jax.numpy as jnp
from jax.experimental import pallas as pl
from jax.experimental.pallas import tpu as pltpu

_N = 10000
_D = 128
_BM = 400
_NB = _N // _BM
_K_STASH = 3


def _proj_body(feat_ref, w1_ref, g_ref):
    f = feat_ref[...].astype(jnp.bfloat16)
    w = w1_ref[...].astype(jnp.bfloat16)
    g = jax.lax.dot_general(
        f, w, (((1,), (0,)), ((), ())), preferred_element_type=jnp.float32
    )
    g_ref[...] = g.astype(jnp.bfloat16)


_CHUNKS = (0, 2560, 5120, 7680, 10000)


def _dot_bf16(a, b):
    return jax.lax.dot_general(
        a, b, (((1,), (0,)), ((), ())), preferred_element_type=jnp.float32
    )


def _fused_body(adj_ref, g_ref, b1_ref, w2_ref, out_ref, h2_ref, stash_ref):
    p = pl.program_id(0)
    i = pl.program_id(1)
    first_stashed = _NB - _K_STASH

    @pl.when(p == 0)
    def _phase0():
        acc = _dot_bf16(adj_ref[...], g_ref[...])
        h = jnp.tanh(acc + b1_ref[...])
        h2 = _dot_bf16(h.astype(jnp.bfloat16), w2_ref[...].astype(jnp.bfloat16))
        h2_ref[pl.ds(i * _BM, _BM), :] = h2.astype(jnp.bfloat16)

        @pl.when(i >= first_stashed)
        def _save():
            for c in range(len(_CHUNKS) - 1):
                lo, hi = _CHUNKS[c], _CHUNKS[c + 1]
                stash_ref[pl.ds((i - first_stashed) * _BM, _BM), lo:hi] = (
                    adj_ref[:, lo:hi].astype(jnp.bfloat16)
                )

    @pl.when(jnp.logical_and(p == 1, i < first_stashed))
    def _phase1_stream():
        out_ref[...] = _dot_bf16(adj_ref[...], h2_ref[...])

    @pl.when(jnp.logical_and(p == 1, i >= first_stashed))
    def _phase1_stash():
        a = stash_ref[pl.ds((i - first_stashed) * _BM, _BM), :]
        out_ref[...] = _dot_bf16(a, h2_ref[...])


def _adj_index(p, i):
    first_stashed = _NB - _K_STASH
    streamed = jnp.where(
        jnp.logical_and(p == 1, i >= first_stashed), first_stashed - 1, i
    )
    return (streamed, 0)


@jax.jit
def _run(adj, feat, W1, b1, W2):
    n, d, bm = _N, _D, _BM

    g = pl.pallas_call(
        _proj_body,
        out_shape=jax.ShapeDtypeStruct((n, d), jnp.bfloat16),
    )(feat, W1)

    b1_2d = b1.reshape(1, d)

    out = pl.pallas_call(
        _fused_body,
        grid=(2, _NB),
        in_specs=[
            pl.BlockSpec((bm, n), _adj_index),
            pl.BlockSpec((n, d), lambda p, i: (0, 0)),
            pl.BlockSpec((1, d), lambda p, i: (0, 0)),
            pl.BlockSpec((d, d), lambda p, i: (0, 0)),
        ],
        out_specs=pl.BlockSpec((bm, d), lambda p, i: (i, 0)),
        out_shape=jax.ShapeDtypeStruct((n, d), jnp.float32),
        scratch_shapes=[
            pltpu.VMEM((n, d), jnp.bfloat16),
            pltpu.VMEM((_K_STASH * bm, n), jnp.bfloat16),
        ],
        compiler_params=pltpu.CompilerParams(
            vmem_limit_bytes=63 * 1024 * 1024,
        ),
    )(adj, g, b1_2d, W2)
    return out


def kernel(adj, feat, W1, b1, W2):
    return _run(adj, feat, W1, b1, W2)
